# baseline (speedup 1.0000x reference)
import functools

import jax
import jax.numpy as jnp
from jax.experimental import pallas as pl


def _prologue_body(fea_ref, B_ref, c0_ref, p01_ref, p2_ref):
    t = (jnp.dot(fea_ref[...], B_ref[...], preferred_element_type=jnp.float32)
         + c0_ref[...])
    p01_ref[...] = t[:, :32]
    p2_ref[...] = t[:, 32:]


def _pass1_body(a_ref, rhs_ref, y1_ref, y2_ref):
    t = jnp.dot(a_ref[...].astype(jnp.bfloat16),
                rhs_ref[...].astype(jnp.bfloat16),
                preferred_element_type=jnp.float32)
    y1_ref[...] = t[:, :16]
    y2_ref[...] = t[:, 16:]


def _pass2_body(a_ref, y1_ref, y2_ref, p2_ref, b_ref, out_ref):
    t = jnp.dot(a_ref[...].astype(jnp.bfloat16),
                y1_ref[...].astype(jnp.bfloat16),
                preferred_element_type=jnp.float32)
    t = t + y2_ref[...] + p2_ref[...] + b_ref[...]
    m = jnp.max(t, axis=1, keepdims=True)
    e = jnp.exp(t - m)
    lse = jnp.log(jnp.sum(e, axis=1, keepdims=True))
    out_ref[...] = t - m - lse


@functools.partial(jax.jit, static_argnames=("blk",))
def _run(fea, adj, B, c0, b_out, blk=400):
    n, nfeat = fea.shape
    nout = B.shape[1]
    grid = n // blk

    P01, P2 = pl.pallas_call(
        _prologue_body,
        grid=(grid,),
        in_specs=[
            pl.BlockSpec((blk, nfeat), lambda i: (i, 0)),
            pl.BlockSpec((nfeat, nout), lambda i: (0, 0)),
            pl.BlockSpec((1, nout), lambda i: (0, 0)),
        ],
        out_specs=[
            pl.BlockSpec((blk, 32), lambda i: (i, 0)),
            pl.BlockSpec((blk, 16), lambda i: (i, 0)),
        ],
        out_shape=[
            jax.ShapeDtypeStruct((n, 32), jnp.float32),
            jax.ShapeDtypeStruct((n, 16), jnp.float32),
        ],
    )(fea, B, c0.reshape(1, -1))

    Y1, Y2 = pl.pallas_call(
        _pass1_body,
        grid=(grid,),
        in_specs=[
            pl.BlockSpec((blk, n), lambda i: (i, 0)),
            pl.BlockSpec((n, 32), lambda i: (0, 0)),
        ],
        out_specs=[
            pl.BlockSpec((blk, 16), lambda i: (i, 0)),
            pl.BlockSpec((blk, 16), lambda i: (i, 0)),
        ],
        out_shape=[
            jax.ShapeDtypeStruct((n, 16), jnp.float32),
            jax.ShapeDtypeStruct((n, 16), jnp.float32),
        ],
    )(adj, P01)

    out = pl.pallas_call(
        _pass2_body,
        grid=(grid,),
        in_specs=[
            pl.BlockSpec((blk, n), lambda i: (i, 0)),
            pl.BlockSpec((n, 16), lambda i: (0, 0)),
            pl.BlockSpec((blk, 16), lambda i: (i, 0)),
            pl.BlockSpec((blk, 16), lambda i: (i, 0)),
            pl.BlockSpec((1, 16), lambda i: (0, 0)),
        ],
        out_specs=pl.BlockSpec((blk, 16), lambda i: (i, 0)),
        out_shape=jax.ShapeDtypeStruct((n, 16), jnp.float32),
    )(adj, Y1, Y2, P2, b_out.reshape(1, -1))

    return out


def kernel(fea, adj, W_in, Wself_in, b_in, W_out, Wself_out, b_out):
    G0 = W_in @ W_out
    G1 = Wself_in @ W_out + W_in @ Wself_out
    G2 = Wself_in @ Wself_out
    B = jnp.concatenate([G0, G1, G2], axis=1)
    c0 = jnp.concatenate([jnp.zeros_like(b_out),
                          b_in @ W_out,
                          b_in @ Wself_out], axis=0)
    return _run(fea, adj, B, c0, b_out)


# speedup vs baseline: 1.0155x; 1.0155x over previous
import functools

import jax
import jax.numpy as jnp
from jax.experimental import pallas as pl


def _prologue_body(fea_ref, B_ref, c0_ref, p01_ref, p2_ref):
    t = (jnp.dot(fea_ref[...], B_ref[...], preferred_element_type=jnp.float32)
         + c0_ref[...])
    p01_ref[...] = t[:, :32]
    p2_ref[...] = t[:, 32:]


def _pass1_body(a_ref, rhs_ref, y1_ref, y2_ref):
    t = jnp.dot(a_ref[...].astype(jnp.bfloat16),
                rhs_ref[...].astype(jnp.bfloat16),
                preferred_element_type=jnp.float32)
    y1_ref[...] = t[:, :16]
    y2_ref[...] = t[:, 16:]


def _pass2_body(a_ref, y1_ref, y2_ref, p2_ref, b_ref, out_ref):
    t = jnp.dot(a_ref[...].astype(jnp.bfloat16),
                y1_ref[...].astype(jnp.bfloat16),
                preferred_element_type=jnp.float32)
    t = t + y2_ref[...] + p2_ref[...] + b_ref[...]
    m = jnp.max(t, axis=1, keepdims=True)
    e = jnp.exp(t - m)
    lse = jnp.log(jnp.sum(e, axis=1, keepdims=True))
    out_ref[...] = t - m - lse


@functools.partial(jax.jit, static_argnames=("blk",))
def _run(fea, adj, B, c0, b_out, blk=640):
    n, nfeat = fea.shape
    nout = B.shape[1]
    grid = (n + blk - 1) // blk

    P01, P2 = pl.pallas_call(
        _prologue_body,
        grid=(grid,),
        in_specs=[
            pl.BlockSpec((blk, nfeat), lambda i: (i, 0)),
            pl.BlockSpec((nfeat, nout), lambda i: (0, 0)),
            pl.BlockSpec((1, nout), lambda i: (0, 0)),
        ],
        out_specs=[
            pl.BlockSpec((blk, 32), lambda i: (i, 0)),
            pl.BlockSpec((blk, 16), lambda i: (i, 0)),
        ],
        out_shape=[
            jax.ShapeDtypeStruct((n, 32), jnp.float32),
            jax.ShapeDtypeStruct((n, 16), jnp.float32),
        ],
    )(fea, B, c0.reshape(1, -1))

    Y1, Y2 = pl.pallas_call(
        _pass1_body,
        grid=(grid,),
        in_specs=[
            pl.BlockSpec((blk, n), lambda i: (i, 0)),
            pl.BlockSpec((n, 32), lambda i: (0, 0)),
        ],
        out_specs=[
            pl.BlockSpec((blk, 16), lambda i: (i, 0)),
            pl.BlockSpec((blk, 16), lambda i: (i, 0)),
        ],
        out_shape=[
            jax.ShapeDtypeStruct((n, 16), jnp.float32),
            jax.ShapeDtypeStruct((n, 16), jnp.float32),
        ],
    )(adj, P01)

    out = pl.pallas_call(
        _pass2_body,
        grid=(grid,),
        in_specs=[
            pl.BlockSpec((blk, n), lambda i: (i, 0)),
            pl.BlockSpec((n, 16), lambda i: (0, 0)),
            pl.BlockSpec((blk, 16), lambda i: (i, 0)),
            pl.BlockSpec((blk, 16), lambda i: (i, 0)),
            pl.BlockSpec((1, 16), lambda i: (0, 0)),
        ],
        out_specs=pl.BlockSpec((blk, 16), lambda i: (i, 0)),
        out_shape=jax.ShapeDtypeStruct((n, 16), jnp.float32),
    )(adj, Y1, Y2, P2, b_out.reshape(1, -1))

    return out


def kernel(fea, adj, W_in, Wself_in, b_in, W_out, Wself_out, b_out):
    G0 = W_in @ W_out
    G1 = Wself_in @ W_out + W_in @ Wself_out
    G2 = Wself_in @ Wself_out
    B = jnp.concatenate([G0, G1, G2], axis=1)
    c0 = jnp.concatenate([jnp.zeros_like(b_out),
                          b_in @ W_out,
                          b_in @ Wself_out], axis=0)
    return _run(fea, adj, B, c0, b_out)
